# baseline (speedup 1.0000x reference)
import functools

import jax
import jax.numpy as jnp
from jax import lax
from jax.experimental import pallas as pl
from jax.experimental.pallas import tpu as pltpu
from jax.experimental.pallas import tpu_sc as plsc

N = 10000
E = 320000
D = 128

NC = 2
NS = 16
NW = NC * NS
EPW = E // NW
CHUNK = 128
NCHUNK = 80
EPW_PAD = NCHUNK * CHUNK
N_PAD = 10240
TRASH_ROW = N
ROWS_PER_TILE = N_PAD // NS

_MESH = plsc.VectorSubcoreMesh(core_axis_name="c", subcore_axis_name="s")


@functools.partial(
    pl.kernel,
    out_type=jax.ShapeDtypeStruct((NC, N_PAD, D), jnp.float32),
    mesh=_MESH,
    scratch_types=[
        pltpu.VMEM((NCHUNK, CHUNK), jnp.int32),
        pltpu.VMEM((NCHUNK, CHUNK), jnp.int32),
        pltpu.VMEM((NCHUNK, CHUNK), jnp.float32),
        pltpu.VMEM((CHUNK, D), jnp.float32),
        pltpu.VMEM_SHARED((N_PAD, D), jnp.float32),
        pltpu.SemaphoreType.DMA,
    ],
)
def _sc_segment_sum(h_hbm, src_hbm, dst_hbm, w_hbm, zeros_hbm, out_hbm,
                    src_v, dst_v, w_v, rows_v, acc_sh, sem):
    c = lax.axis_index("c")
    s = lax.axis_index("s")
    wid = s * NC + c

    pltpu.sync_copy(zeros_hbm,
                    acc_sh.at[pl.ds(s * ROWS_PER_TILE, ROWS_PER_TILE)])
    pltpu.sync_copy(src_hbm.at[wid], src_v)
    pltpu.sync_copy(dst_hbm.at[wid], dst_v)
    pltpu.sync_copy(w_hbm.at[wid], w_v)
    plsc.subcore_barrier()

    def chunk_body(j, carry):
        pltpu.async_copy(h_hbm.at[src_v.at[j]], rows_v, sem).wait()

        def group_body(g, carry2):
            w16 = w_v[j, pl.ds(g * 16, 16)]
            e0 = g * 16
            for el in range(16):
                wv = jnp.full((16,), w16[el], dtype=jnp.float32)
                for t in range(D // 16):
                    sl = pl.ds(t * 16, 16)
                    rows_v[e0 + el, sl] = rows_v[e0 + el, sl] * wv
            return carry2

        lax.fori_loop(0, CHUNK // 16, group_body, 0)

        pltpu.sync_copy(rows_v, acc_sh.at[dst_v.at[j]], add=True)
        return carry

    lax.fori_loop(0, NCHUNK, chunk_body, 0)
    plsc.subcore_barrier()

    pltpu.sync_copy(acc_sh.at[pl.ds(s * ROWS_PER_TILE, ROWS_PER_TILE)],
                    out_hbm.at[c, pl.ds(s * ROWS_PER_TILE, ROWS_PER_TILE)])


def _tc_body(relu, p_ref, h_ref, wr_ref, br_ref, wo_ref, a_ref, b_ref, o_ref):
    agg = p_ref[0, :N] + p_ref[1, :N]
    out = jnp.dot(agg, wr_ref[...], preferred_element_type=jnp.float32)
    out = out + jnp.dot(h_ref[...], wo_ref[...],
                        preferred_element_type=jnp.float32)
    out = out + br_ref[...]
    mean = jnp.mean(out, axis=-1, keepdims=True)
    cent = out - mean
    var = jnp.sum(cent * cent, axis=-1, keepdims=True) / (D - 1)
    y = a_ref[...] * cent / (jnp.sqrt(var) + 1e-6) + b_ref[...]
    if relu:
        y = jnp.maximum(y, 0.0)
    o_ref[...] = y


def _tc_stage(p, h, W_rel, b_rel, W_root, a, b, relu):
    return pl.pallas_call(
        functools.partial(_tc_body, relu),
        out_shape=jax.ShapeDtypeStruct((N, D), jnp.float32),
    )(p, h, W_rel, b_rel.reshape(1, D), W_root, a.reshape(1, D),
      b.reshape(1, D))


def kernel(x, edge_index, edge_weight,
           W_rel0, b_rel0, W_root0, a0, b0,
           W_rel1, b_rel1, W_root1, a1, b1,
           W_rel2, b_rel2, W_root2, af, bf):
    pad = EPW_PAD - EPW
    src = jnp.pad(edge_index[0].astype(jnp.int32).reshape(NW, EPW),
                  ((0, 0), (0, pad))).reshape(NW, NCHUNK, CHUNK)
    dst = jnp.pad(edge_index[1].astype(jnp.int32).reshape(NW, EPW),
                  ((0, 0), (0, pad)),
                  constant_values=TRASH_ROW).reshape(NW, NCHUNK, CHUNK)
    w = jnp.pad(edge_weight.reshape(NW, EPW),
                ((0, 0), (0, pad))).reshape(NW, NCHUNK, CHUNK)
    zeros = jnp.zeros((ROWS_PER_TILE, D), jnp.float32)

    h = x
    for (W_rel, b_rel, W_root, a, b, relu) in (
            (W_rel0, b_rel0, W_root0, a0, b0, True),
            (W_rel1, b_rel1, W_root1, a1, b1, True),
            (W_rel2, b_rel2, W_root2, af, bf, False)):
        p = _sc_segment_sum(h, src, dst, w, zeros)
        h = _tc_stage(p, h, W_rel, b_rel, W_root, a, b, relu)
    return h


# speedup vs baseline: 1.4137x; 1.4137x over previous
import functools

import jax
import jax.numpy as jnp
from jax import lax
from jax.experimental import pallas as pl
from jax.experimental.pallas import tpu as pltpu
from jax.experimental.pallas import tpu_sc as plsc

N = 10000
E = 320000
D = 128

NC = 2
NS = 16
NW = NC * NS
EPW = E // NW
CHUNK = 128
NCHUNK = 79
EPW_PAD = NCHUNK * CHUNK
N_PAD = 10240
TRASH_ROW = N
ROWS_PER_TILE = N_PAD // NS

_MESH = plsc.VectorSubcoreMesh(core_axis_name="c", subcore_axis_name="s")


@functools.partial(
    pl.kernel,
    out_type=jax.ShapeDtypeStruct((NC, N_PAD, D), jnp.float32),
    mesh=_MESH,
    scratch_types=[
        pltpu.VMEM((NCHUNK, CHUNK), jnp.int32),
        pltpu.VMEM((NCHUNK, CHUNK), jnp.int32),
        pltpu.VMEM((NCHUNK, CHUNK), jnp.float32),
        pltpu.VMEM((CHUNK, D), jnp.float32),
        pltpu.VMEM_SHARED((N_PAD, D), jnp.float32),
        pltpu.SemaphoreType.DMA,
    ],
)
def _sc_segment_sum(h_hbm, src_hbm, dst_hbm, w_hbm, zeros_hbm, out_hbm,
                    src_v, dst_v, w_v, rows_v, acc_sh, sem):
    c = lax.axis_index("c")
    s = lax.axis_index("s")
    wid = s * NC + c

    pltpu.sync_copy(zeros_hbm,
                    acc_sh.at[pl.ds(s * ROWS_PER_TILE, ROWS_PER_TILE)])
    pltpu.sync_copy(src_hbm.at[wid], src_v)
    pltpu.sync_copy(dst_hbm.at[wid], dst_v)
    pltpu.sync_copy(w_hbm.at[wid], w_v)
    plsc.subcore_barrier()

    def chunk_body(j, carry):
        pltpu.async_copy(h_hbm.at[src_v.at[j]], rows_v, sem).wait()

        def group_body(g, carry2):
            w16 = w_v[j, pl.ds(g * 16, 16)]
            e0 = g * 16
            for el in range(16):
                wv = jnp.full((16,), w16[el], dtype=jnp.float32)
                for t in range(D // 16):
                    sl = pl.ds(t * 16, 16)
                    rows_v[e0 + el, sl] = rows_v[e0 + el, sl] * wv
            return carry2

        lax.fori_loop(0, CHUNK // 16, group_body, 0)

        pltpu.sync_copy(rows_v, acc_sh.at[dst_v.at[j]], add=True)
        return carry

    lax.fori_loop(0, NCHUNK, chunk_body, 0)
    plsc.subcore_barrier()

    pltpu.sync_copy(acc_sh.at[pl.ds(s * ROWS_PER_TILE, ROWS_PER_TILE)],
                    out_hbm.at[c, pl.ds(s * ROWS_PER_TILE, ROWS_PER_TILE)])


def _tc_body(relu, p_ref, h_ref, wr_ref, br_ref, wo_ref, a_ref, b_ref, o_ref):
    agg = p_ref[0, :N] + p_ref[1, :N]
    out = jnp.dot(agg, wr_ref[...], preferred_element_type=jnp.float32)
    out = out + jnp.dot(h_ref[...], wo_ref[...],
                        preferred_element_type=jnp.float32)
    out = out + br_ref[...]
    mean = jnp.mean(out, axis=-1, keepdims=True)
    cent = out - mean
    var = jnp.sum(cent * cent, axis=-1, keepdims=True) / (D - 1)
    y = a_ref[...] * cent / (jnp.sqrt(var) + 1e-6) + b_ref[...]
    if relu:
        y = jnp.maximum(y, 0.0)
    o_ref[...] = y


def _tc_stage(p, h, W_rel, b_rel, W_root, a, b, relu):
    return pl.pallas_call(
        functools.partial(_tc_body, relu),
        out_shape=jax.ShapeDtypeStruct((N, D), jnp.float32),
    )(p, h, W_rel, b_rel.reshape(1, D), W_root, a.reshape(1, D),
      b.reshape(1, D))


def kernel(x, edge_index, edge_weight,
           W_rel0, b_rel0, W_root0, a0, b0,
           W_rel1, b_rel1, W_root1, a1, b1,
           W_rel2, b_rel2, W_root2, af, bf):
    pad = EPW_PAD - EPW
    src = jnp.pad(edge_index[0].astype(jnp.int32).reshape(NW, EPW),
                  ((0, 0), (0, pad))).reshape(NW, NCHUNK, CHUNK)
    pad_dst = TRASH_ROW + jnp.arange(pad, dtype=jnp.int32) % (N_PAD - N)
    dst = jnp.concatenate(
        [edge_index[1].astype(jnp.int32).reshape(NW, EPW),
         jnp.broadcast_to(pad_dst, (NW, pad))],
        axis=1).reshape(NW, NCHUNK, CHUNK)
    w = jnp.pad(edge_weight.reshape(NW, EPW),
                ((0, 0), (0, pad))).reshape(NW, NCHUNK, CHUNK)
    zeros = jnp.zeros((ROWS_PER_TILE, D), jnp.float32)

    h = x
    for (W_rel, b_rel, W_root, a, b, relu) in (
            (W_rel0, b_rel0, W_root0, a0, b0, True),
            (W_rel1, b_rel1, W_root1, a1, b1, True),
            (W_rel2, b_rel2, W_root2, af, bf, False)):
        p = _sc_segment_sum(h, src, dst, w, zeros)
        h = _tc_stage(p, h, W_rel, b_rel, W_root, a, b, relu)
    return h
